# baseline (speedup 1.0000x reference)
import functools

import jax
import jax.numpy as jnp
from jax.experimental import pallas as pl
from jax.experimental.pallas import tpu as pltpu


_MRB_M = 512


def _layer1_partial_kernel(x_ref, w1_ref, adj_ref, o_ref, s1_ref, *,
                           n_pro, tm_pro, tm):
    t = pl.program_id(1)

    @pl.when(t < n_pro)
    def _prologue():
        for m in range(tm_pro // _MRB_M):
            row = t * tm_pro + m * _MRB_M
            xb = x_ref[pl.ds(m * _MRB_M, _MRB_M), :].astype(jnp.bfloat16)
            s1_ref[pl.ds(row, _MRB_M), :] = jnp.dot(
                xb, w1_ref[...], preferred_element_type=jnp.float32
            ).astype(s1_ref.dtype)

    @pl.when(t >= n_pro)
    def _contract():
        s1 = s1_ref[...]
        for m in range(tm // _MRB_M):
            acc = jnp.dot(
                adj_ref[pl.ds(m * _MRB_M, _MRB_M), :], s1,
                preferred_element_type=jnp.float32,
            )
            o_ref[0, pl.ds(m * _MRB_M, _MRB_M), :] = acc.astype(o_ref.dtype)


def _hidden_kernel(p_ref, b1_ref, w2_ref, o_ref):
    h32 = (p_ref[0].astype(jnp.float32) + p_ref[1].astype(jnp.float32)
           + b1_ref[...])
    h = jnp.maximum(h32, 0.0).astype(jnp.bfloat16)
    for m in range(h.shape[0] // _MRB_M):
        sl = slice(m * _MRB_M, (m + 1) * _MRB_M)
        o_ref[sl, :] = jnp.dot(
            h[sl, :], w2_ref[...], preferred_element_type=jnp.float32
        ).astype(o_ref.dtype)


def _layer2_kernel(s2_ref, adj_ref, b2_ref, o_ref):
    acc = jnp.dot(adj_ref[...], s2_ref[...], preferred_element_type=jnp.float32)
    o_ref[...] = acc + b2_ref[...]


@jax.jit
def _forward(x, adj_p, w1_p, b1_p, w2_p, b2_p):
    Np = adj_p.shape[0]
    D = x.shape[1]
    Z = w1_p.shape[1]
    C = w2_p.shape[1]
    cd = jnp.bfloat16

    NC = 2
    H = Np // NC
    TM_PRO = 1024
    TM = 1024
    TMH = 1024
    TM2 = 1024

    n_pro = H // TM_PRO
    n_con = Np // TM

    partial = pl.pallas_call(
        functools.partial(_layer1_partial_kernel, n_pro=n_pro,
                          tm_pro=TM_PRO, tm=TM),
        out_shape=jax.ShapeDtypeStruct((NC, Np, Z), cd),
        grid_spec=pltpu.PrefetchScalarGridSpec(
            num_scalar_prefetch=0,
            grid=(NC, n_pro + n_con),
            in_specs=[
                pl.BlockSpec(
                    (TM_PRO, D),
                    lambda c, t: (c * (H // TM_PRO) + jnp.minimum(t, (H // TM_PRO) - 1), 0)),
                pl.BlockSpec((D, Z), lambda c, t: (0, 0)),
                pl.BlockSpec(
                    (TM, H),
                    lambda c, t: (jnp.maximum(t - (H // TM_PRO), 0), c)),
            ],
            out_specs=pl.BlockSpec(
                (1, TM, Z),
                lambda c, t: (c, jnp.maximum(t - (H // TM_PRO), 0), 0)),
            scratch_shapes=[pltpu.VMEM((H, Z), cd)],
        ),
        compiler_params=pltpu.CompilerParams(
            dimension_semantics=("parallel", "arbitrary")),
    )(x, w1_p, adj_p)

    s2 = pl.pallas_call(
        _hidden_kernel,
        out_shape=jax.ShapeDtypeStruct((Np, C), cd),
        grid_spec=pltpu.PrefetchScalarGridSpec(
            num_scalar_prefetch=0,
            grid=(Np // TMH,),
            in_specs=[
                pl.BlockSpec((NC, TMH, Z), lambda i: (0, i, 0)),
                pl.BlockSpec((1, Z), lambda i: (0, 0)),
                pl.BlockSpec((Z, C), lambda i: (0, 0)),
            ],
            out_specs=pl.BlockSpec((TMH, C), lambda i: (i, 0)),
        ),
        compiler_params=pltpu.CompilerParams(dimension_semantics=("parallel",)),
    )(partial, b1_p, w2_p)

    out = pl.pallas_call(
        _layer2_kernel,
        out_shape=jax.ShapeDtypeStruct((Np, C), jnp.float32),
        grid_spec=pltpu.PrefetchScalarGridSpec(
            num_scalar_prefetch=0,
            grid=(Np // TM2,),
            in_specs=[
                pl.BlockSpec((Np, C), lambda i: (0, 0)),
                pl.BlockSpec((TM2, Np), lambda i: (i, 0)),
                pl.BlockSpec((1, C), lambda i: (0, 0)),
            ],
            out_specs=pl.BlockSpec((TM2, C), lambda i: (i, 0)),
        ),
        compiler_params=pltpu.CompilerParams(dimension_semantics=("parallel",)),
    )(s2, adj_p, b2_p)

    return out


def kernel(x, adj_p, w1_p, b1_p, w2_p, b2_p):
    N = x.shape[0]
    C = w2_p.shape[1]
    out = _forward(x, adj_p, w1_p, b1_p, w2_p, b2_p)
    return out[:N, :C]


# speedup vs baseline: 1.0314x; 1.0314x over previous
import functools

import jax
import jax.numpy as jnp
from jax.experimental import pallas as pl
from jax.experimental.pallas import tpu as pltpu


_MRB_M = 512


def _support1_kernel(x_ref, w1_ref, o_ref):
    for m in range(x_ref.shape[0] // _MRB_M):
        sl = slice(m * _MRB_M, (m + 1) * _MRB_M)
        x = x_ref[sl, :].astype(jnp.bfloat16)
        o_ref[sl, :] = jnp.dot(
            x, w1_ref[...], preferred_element_type=jnp.float32
        ).astype(o_ref.dtype)


def _layer1_kernel(s1_ref, adj_ref, b1_ref, w2_ref, o_ref):
    for m in range(adj_ref.shape[0] // _MRB_M):
        sl = slice(m * _MRB_M, (m + 1) * _MRB_M)
        acc = jnp.dot(
            adj_ref[sl, :], s1_ref[...], preferred_element_type=jnp.float32
        )
        h = jnp.maximum(acc + b1_ref[...], 0.0).astype(jnp.bfloat16)
        o_ref[sl, :] = jnp.dot(
            h, w2_ref[...], preferred_element_type=jnp.float32
        ).astype(o_ref.dtype)


def _layer2_kernel(s2_ref, adj_ref, b2_ref, o_ref):
    acc = jnp.dot(adj_ref[...], s2_ref[...], preferred_element_type=jnp.float32)
    o_ref[...] = acc + b2_ref[...]


@jax.jit
def _forward(x, adj_p, w1_p, b1_p, w2_p, b2_p):
    Np = adj_p.shape[0]
    D = x.shape[1]
    Z = w1_p.shape[1]
    C = w2_p.shape[1]
    cd = jnp.bfloat16

    TM1 = 2048
    TM = 1024
    TM2 = 1024

    s1 = pl.pallas_call(
        _support1_kernel,
        out_shape=jax.ShapeDtypeStruct((Np, Z), cd),
        grid_spec=pltpu.PrefetchScalarGridSpec(
            num_scalar_prefetch=0,
            grid=(Np // TM1,),
            in_specs=[
                pl.BlockSpec((TM1, D), lambda i: (i, 0)),
                pl.BlockSpec((D, Z), lambda i: (0, 0)),
            ],
            out_specs=pl.BlockSpec((TM1, Z), lambda i: (i, 0)),
        ),
        compiler_params=pltpu.CompilerParams(dimension_semantics=("parallel",)),
    )(x, w1_p)
    s2 = pl.pallas_call(
        _layer1_kernel,
        out_shape=jax.ShapeDtypeStruct((Np, C), cd),
        grid_spec=pltpu.PrefetchScalarGridSpec(
            num_scalar_prefetch=0,
            grid=(Np // TM,),
            in_specs=[
                pl.BlockSpec((Np, Z), lambda i: (0, 0)),
                pl.BlockSpec((TM, Np), lambda i: (i, 0)),
                pl.BlockSpec((1, Z), lambda i: (0, 0)),
                pl.BlockSpec((Z, C), lambda i: (0, 0)),
            ],
            out_specs=pl.BlockSpec((TM, C), lambda i: (i, 0)),
        ),
        compiler_params=pltpu.CompilerParams(dimension_semantics=("parallel",)),
    )(s1, adj_p, b1_p, w2_p)

    out = pl.pallas_call(
        _layer2_kernel,
        out_shape=jax.ShapeDtypeStruct((Np, C), jnp.float32),
        grid_spec=pltpu.PrefetchScalarGridSpec(
            num_scalar_prefetch=0,
            grid=(Np // TM2,),
            in_specs=[
                pl.BlockSpec((Np, C), lambda i: (0, 0)),
                pl.BlockSpec((TM2, Np), lambda i: (i, 0)),
                pl.BlockSpec((1, C), lambda i: (0, 0)),
            ],
            out_specs=pl.BlockSpec((TM2, C), lambda i: (i, 0)),
        ),
        compiler_params=pltpu.CompilerParams(dimension_semantics=("parallel",)),
    )(s2, adj_p, b2_p)

    return out


def kernel(x, adj_p, w1_p, b1_p, w2_p, b2_p):
    N = x.shape[0]
    C = w2_p.shape[1]
    out = _forward(x, adj_p, w1_p, b1_p, w2_p, b2_p)
    return out[:N, :C]
